# baseline (speedup 1.0000x reference)
import functools
import math

import jax
import jax.numpy as jnp
from jax import lax
from jax.experimental import pallas as pl
from jax.experimental.pallas import tpu as pltpu
from jax.experimental.pallas import tpu_sc as plsc

SEQ_LEN = 8192
DIM = 64
_GRAN = 16
_NC, _NS = 2, 16
_NW = _NC * _NS
_B_PER_W = SEQ_LEN // _NW
_K = 16
_N_BATCH = _B_PER_W // _K


def _positional_encoding_t():
    position = jnp.arange(SEQ_LEN, dtype=jnp.float32)[None, :]
    div_term = jnp.exp(
        jnp.arange(0, DIM, 2, dtype=jnp.float32) * (-math.log(10000.0) / DIM)
    )
    pe = jnp.zeros((DIM, SEQ_LEN), dtype=jnp.float32)
    pe = pe.at[0::2, :].set(jnp.sin(div_term[:, None] * position))
    pe = pe.at[1::2, :].set(jnp.cos(div_term[:, None] * position))
    return pe


_mesh = plsc.VectorSubcoreMesh(core_axis_name="c", subcore_axis_name="s")


@functools.partial(
    pl.kernel,
    mesh=_mesh,
    out_type=jax.ShapeDtypeStruct((DIM, SEQ_LEN), jnp.float32),
    scratch_types=[
        pltpu.VMEM((_B_PER_W,), jnp.int32),
        pltpu.VMEM((_K, DIM, _GRAN), jnp.float32),
        pltpu.VMEM((DIM, _B_PER_W), jnp.float32),
        pltpu.SemaphoreType.DMA,
    ],
    compiler_params=pltpu.CompilerParams(
        use_tc_tiling_on_sc=False, needs_layout_passes=False
    ),
)
def _sc_gather_t(table_t_hbm, idx_hbm, out_hbm, idx_v, ring_v, out_v, sem):
    wid = lax.axis_index("s") * _NC + lax.axis_index("c")
    base = wid * _B_PER_W
    pltpu.sync_copy(idx_hbm.at[pl.ds(base, _B_PER_W)], idx_v)

    @pl.loop(0, _N_BATCH)
    def _(batch):
        jbase = pl.multiple_of(batch * _K, _K)
        idx_vec = idx_v[pl.ds(jbase, _K)]
        for b in range(_K):
            s = idx_vec[b]
            goff = pl.multiple_of((s >> 4) << 4, _GRAN)
            pltpu.async_copy(
                table_t_hbm.at[:, pl.ds(goff, _GRAN)], ring_v.at[b], sem
            )

        @pl.loop(0, _K)
        def _(b):
            pltpu.make_async_copy(
                table_t_hbm.at[:, pl.ds(0, _GRAN)], ring_v.at[b], sem
            ).wait()

        lane_vec = idx_vec & (_GRAN - 1)
        for b in range(_K):
            lane_v = jnp.full((_GRAN,), lane_vec[b], jnp.int32)
            col_v = jnp.full((_GRAN,), jbase + b, jnp.int32)
            for r in range(DIM // _GRAN):
                rid = lax.broadcasted_iota(jnp.int32, (_GRAN,), 0) + r * _GRAN
                vals = plsc.load_gather(ring_v.at[b], [rid, lane_v])
                plsc.store_scatter(out_v, [rid, col_v], vals)

    pltpu.sync_copy(out_v, out_hbm.at[:, pl.ds(base, _B_PER_W)])


def _tc_add(x_ref, pe_ref, o_ref):
    o_ref[...] = x_ref[...] + pe_ref[...]


def kernel(indices, table):
    idx = indices.astype(jnp.int32)
    gathered_t = _sc_gather_t(table.T, idx)
    pe_t = _positional_encoding_t()
    out_t = pl.pallas_call(
        _tc_add,
        out_shape=jax.ShapeDtypeStruct((DIM, SEQ_LEN), jnp.float32),
    )(gathered_t, pe_t)
    return out_t.T[None, :, :]


# speedup vs baseline: 5.7771x; 5.7771x over previous
import functools
import math

import jax
import jax.numpy as jnp
from jax import lax
from jax.experimental import pallas as pl
from jax.experimental.pallas import tpu as pltpu
from jax.experimental.pallas import tpu_sc as plsc

SEQ_LEN = 8192
DIM = 64
VOCAB = 1000000
_NC, _NS = 2, 16
_NW = _NC * _NS
_B_PER_W = SEQ_LEN // _NW
_TBLOCK = 2048


def _positional_encoding_t():
    position = jnp.arange(SEQ_LEN, dtype=jnp.float32)[None, :]
    div_term = jnp.exp(
        jnp.arange(0, DIM, 2, dtype=jnp.float32) * (-math.log(10000.0) / DIM)
    )
    pe = jnp.zeros((DIM, SEQ_LEN), dtype=jnp.float32)
    pe = pe.at[0::2, :].set(jnp.sin(div_term[:, None] * position))
    pe = pe.at[1::2, :].set(jnp.cos(div_term[:, None] * position))
    return pe


def _tc_transpose(x_ref, o_ref):
    o_ref[...] = x_ref[...].T


_mesh = plsc.VectorSubcoreMesh(core_axis_name="c", subcore_axis_name="s")


@functools.partial(
    pl.kernel,
    mesh=_mesh,
    out_type=jax.ShapeDtypeStruct((SEQ_LEN, DIM), jnp.float32),
    scratch_types=[
        pltpu.VMEM((_B_PER_W,), jnp.int32),
        pltpu.VMEM((_B_PER_W, DIM), jnp.float32),
        pltpu.SemaphoreType.DMA,
    ],
    compiler_params=pltpu.CompilerParams(use_tc_tiling_on_sc=False),
)
def _sc_gather(table_hbm, idx_hbm, out_hbm, idx_v, rows_v, sem):
    wid = lax.axis_index("s") * _NC + lax.axis_index("c")
    base = wid * _B_PER_W
    pltpu.sync_copy(idx_hbm.at[pl.ds(base, _B_PER_W)], idx_v)
    pltpu.async_copy(table_hbm.at[idx_v], rows_v, sem).wait()
    pltpu.sync_copy(rows_v, out_hbm.at[pl.ds(base, _B_PER_W)])


def _tc_add_t(x_ref, pe_ref, o_ref):
    o_ref[...] = x_ref[...].T + pe_ref[...]


def kernel(indices, table):
    idx = indices.astype(jnp.int32)
    table_rm = pl.pallas_call(
        _tc_transpose,
        grid=(pl.cdiv(VOCAB, _TBLOCK),),
        in_specs=[pl.BlockSpec((DIM, _TBLOCK), lambda b: (0, b))],
        out_specs=pl.BlockSpec((_TBLOCK, DIM), lambda b: (b, 0)),
        out_shape=jax.ShapeDtypeStruct((VOCAB, DIM), jnp.float32),
    )(table.T)
    gathered = _sc_gather(table_rm, idx)
    pe_t = _positional_encoding_t()
    out_t = pl.pallas_call(
        _tc_add_t,
        out_shape=jax.ShapeDtypeStruct((DIM, SEQ_LEN), jnp.float32),
    )(gathered, pe_t)
    return out_t.T[None, :, :]


# speedup vs baseline: 7.4931x; 1.2970x over previous
import functools
import math

import jax
import jax.numpy as jnp
from jax import lax
from jax.experimental import pallas as pl
from jax.experimental.pallas import tpu as pltpu
from jax.experimental.pallas import tpu_sc as plsc

SEQ_LEN = 8192
DIM = 64
VOCAB = 1000000
_NC, _NS = 2, 16
_NW = _NC * _NS
_B_PER_W = SEQ_LEN // _NW
_TBLOCK = 8192


def _positional_encoding_t():
    position = jnp.arange(SEQ_LEN, dtype=jnp.float32)[None, :]
    div_term = jnp.exp(
        jnp.arange(0, DIM, 2, dtype=jnp.float32) * (-math.log(10000.0) / DIM)
    )
    pe = jnp.zeros((DIM, SEQ_LEN), dtype=jnp.float32)
    pe = pe.at[0::2, :].set(jnp.sin(div_term[:, None] * position))
    pe = pe.at[1::2, :].set(jnp.cos(div_term[:, None] * position))
    return pe


def _tc_transpose(x_ref, o_ref):
    o_ref[...] = x_ref[...].T


_mesh = plsc.VectorSubcoreMesh(core_axis_name="c", subcore_axis_name="s")


@functools.partial(
    pl.kernel,
    mesh=_mesh,
    out_type=jax.ShapeDtypeStruct((SEQ_LEN, DIM), jnp.float32),
    scratch_types=[
        pltpu.VMEM((_B_PER_W,), jnp.int32),
        pltpu.VMEM((_B_PER_W, DIM), jnp.float32),
        pltpu.SemaphoreType.DMA,
    ],
    compiler_params=pltpu.CompilerParams(use_tc_tiling_on_sc=False),
)
def _sc_gather(table_hbm, idx_hbm, out_hbm, idx_v, rows_v, sem):
    wid = lax.axis_index("s") * _NC + lax.axis_index("c")
    base = wid * _B_PER_W
    pltpu.sync_copy(idx_hbm.at[pl.ds(base, _B_PER_W)], idx_v)
    pltpu.async_copy(table_hbm.at[idx_v], rows_v, sem).wait()
    pltpu.sync_copy(rows_v, out_hbm.at[pl.ds(base, _B_PER_W)])


def _tc_add_t(x_ref, pe_ref, o_ref):
    o_ref[...] = x_ref[...].T + pe_ref[...]


def kernel(indices, table):
    idx = indices.astype(jnp.int32)
    table_rm = pl.pallas_call(
        _tc_transpose,
        grid=(pl.cdiv(VOCAB, _TBLOCK),),
        in_specs=[pl.BlockSpec((DIM, _TBLOCK), lambda b: (0, b))],
        out_specs=pl.BlockSpec((_TBLOCK, DIM), lambda b: (b, 0)),
        out_shape=jax.ShapeDtypeStruct((VOCAB, DIM), jnp.float32),
    )(table.T)
    gathered = _sc_gather(table_rm, idx)
    pe_t = _positional_encoding_t()
    out_t = pl.pallas_call(
        _tc_add_t,
        out_shape=jax.ShapeDtypeStruct((DIM, SEQ_LEN), jnp.float32),
    )(gathered, pe_t)
    return out_t.T[None, :, :]
